# baseline (speedup 1.0000x reference)
import dataclasses
import functools

import jax
import jax.numpy as jnp
from jax import lax
from jax.experimental import pallas as pl
from jax.experimental.pallas import tpu as pltpu
from jax.experimental.pallas import tpu_sc as plsc

B, N, D = 4, 8192, 768
TASK_DIM = 64
PHASE_DIV = 2.0
BN = 2048
K0, K1 = 512, 256
NW = 32


def _score_body(x_ref, temb0_ref, temb1_ref, w0_ref, w1_ref, b0_ref, b1_ref,
                s0_ref, s1_ref):
    xb = x_ref[0]
    t0 = jnp.broadcast_to(temb0_ref[0], (BN, TASK_DIM))
    t1 = jnp.broadcast_to(temb1_ref[0], (BN, TASK_DIM))
    h0 = jnp.concatenate([xb, t0], axis=-1)
    h1 = jnp.concatenate([xb, t1], axis=-1)
    s0 = h0 @ w0_ref[...] + b0_ref[...]
    s1 = h1 @ w1_ref[...] + b1_ref[...]
    s0_ref[0, 0, :] = s0[:, 0]
    s1_ref[0, 0, :] = s1[:, 0]


def _scores(x, temb0, temb1, Ws0, bs0, Ws1, bs1):
    grid = (B, N // BN)
    s0, s1 = pl.pallas_call(
        _score_body,
        grid=grid,
        in_specs=[
            pl.BlockSpec((1, BN, D), lambda b, n: (b, n, 0)),
            pl.BlockSpec((1, 1, TASK_DIM), lambda b, n: (b, 0, 0)),
            pl.BlockSpec((1, 1, TASK_DIM), lambda b, n: (b, 0, 0)),
            pl.BlockSpec((D + TASK_DIM, 1), lambda b, n: (0, 0)),
            pl.BlockSpec((D + TASK_DIM, 1), lambda b, n: (0, 0)),
            pl.BlockSpec((1,), lambda b, n: (0,)),
            pl.BlockSpec((1,), lambda b, n: (0,)),
        ],
        out_specs=[
            pl.BlockSpec((1, 1, BN), lambda b, n: (b, 0, n)),
            pl.BlockSpec((1, 1, BN), lambda b, n: (b, 0, n)),
        ],
        out_shape=[
            jax.ShapeDtypeStruct((B, 1, N), jnp.float32),
            jax.ShapeDtypeStruct((B, 1, N), jnp.float32),
        ],
    )(x, temb0.reshape(B, 1, TASK_DIM), temb1.reshape(B, 1, TASK_DIM),
      Ws0, Ws1, bs0, bs1)
    return s0, s1


def _thresh_body(s0_ref, s1_ref, tkey_ref, tbud_ref):
    rows = []
    for s_ref, k in ((s0_ref, K0), (s1_ref, K1)):
        s = s_ref[:, 0, :]
        bits = lax.bitcast_convert_type(s, jnp.uint32)
        ukey = jnp.where(s >= 0, bits | jnp.uint32(0x80000000), ~bits)
        t = jnp.zeros((B, 1), jnp.uint32)
        for bit in range(31, -1, -1):
            cand = t | jnp.uint32(1 << bit)
            cnt = jnp.sum((ukey >= cand).astype(jnp.int32), axis=1,
                          keepdims=True)
            t = jnp.where(cnt >= k, cand, t)
        cgt = jnp.sum((ukey > t).astype(jnp.int32), axis=1, keepdims=True)
        skey = lax.bitcast_convert_type(t ^ jnp.uint32(0x80000000), jnp.int32)
        rows.append((skey, k - cgt))
    tkey = jnp.concatenate([rows[0][0], rows[1][0]], axis=0)
    tbud = jnp.concatenate([rows[0][1], rows[1][1]], axis=0)
    tkey_ref[...] = jnp.broadcast_to(tkey, (2 * B, 16))
    tbud_ref[...] = jnp.broadcast_to(tbud, (2 * B, 16))


def _threshold(s0, s1):
    return pl.pallas_call(
        _thresh_body,
        out_shape=[
            jax.ShapeDtypeStruct((2 * B, 16), jnp.int32),
            jax.ShapeDtypeStruct((2 * B, 16), jnp.int32),
        ],
    )(s0, s1)


def _skey16(v):
    b = lax.bitcast_convert_type(v, jnp.int32)
    return b ^ ((b >> 31) & jnp.int32(0x7FFFFFFF))


def _sc_select_gather(scores_stk, tkey, tbud, x_flat):
    mesh = plsc.VectorSubcoreMesh(core_axis_name="c", subcore_axis_name="s")
    r0, r1 = K0 * B // NW, K1 * B // NW
    cp = pltpu.CompilerParams()
    if "needs_layout_passes" in pltpu.CompilerParams.__dataclass_fields__:
        cp = dataclasses.replace(cp, needs_layout_passes=False)

    @functools.partial(
        pl.kernel, mesh=mesh, compiler_params=cp,
        out_type=[
            jax.ShapeDtypeStruct((B * K0,), jnp.float32),
            jax.ShapeDtypeStruct((B * K1,), jnp.float32),
            jax.ShapeDtypeStruct((B * K0,), jnp.int32),
            jax.ShapeDtypeStruct((B * K1,), jnp.int32),
            jax.ShapeDtypeStruct((B * K0, D), jnp.float32),
            jax.ShapeDtypeStruct((B * K1, D), jnp.float32),
        ],
        scratch_types=[
            pltpu.VMEM((N,), jnp.float32),
            pltpu.VMEM((16,), jnp.int32),
            pltpu.VMEM((16,), jnp.int32),
            pltpu.VMEM((K0,), jnp.int32),
            pltpu.VMEM((K0,), jnp.int32),
            pltpu.VMEM((K0,), jnp.float32),
            pltpu.VMEM((K0,), jnp.int32),
            pltpu.VMEM((K0,), jnp.float32),
            pltpu.VMEM((r0,), jnp.int32),
            pltpu.VMEM((r1,), jnp.int32),
            pltpu.VMEM((r0, D), jnp.float32),
            pltpu.VMEM((r1, D), jnp.float32),
            pltpu.SemaphoreType.DMA,
        ],
    )
    def sck(scores_hbm, tkey_hbm, tbud_hbm, xflat_hbm,
            sc0_hbm, sc1_hbm, if0_hbm, if1_hbm, g0_hbm, g1_hbm,
            scores_v, tk_v, tb_v, ckey_v, cpos_v, cscr_v, spos_v, sscr_v,
            i0_v, i1_v, rows0_v, rows1_v, sem):
        sid = lax.axis_index("s")
        wid = sid * 2 + lax.axis_index("c")
        iota = lax.iota(jnp.int32, 16)

        def unit(k, row, b, sc_out, if_out):
            pltpu.sync_copy(scores_hbm.at[row], scores_v)
            pltpu.sync_copy(tkey_hbm.at[row], tk_v)
            pltpu.sync_copy(tbud_hbm.at[row], tb_v)
            tkey_vec = tk_v[...]
            bud_vec = tb_v[...]

            def comp_body(n, carry):
                cur, eqs = carry
                v = scores_v[pl.ds(n * 16, 16)]
                key = _skey16(v)
                m_gt = key > tkey_vec
                m_eq = key == tkey_vec
                psum = jnp.cumsum(m_eq.astype(jnp.int32))
                sel = m_gt | (m_eq & ((psum + eqs) <= bud_vec))
                seli = sel.astype(jnp.int32)
                dest = (jnp.cumsum(seli) - seli) + cur
                posv = iota + (n * 16 + b * N)
                plsc.store_scatter(ckey_v, [dest], key, mask=sel)
                plsc.store_scatter(cpos_v, [dest], posv, mask=sel)
                plsc.store_scatter(cscr_v, [dest], v, mask=sel)
                return cur + jnp.sum(seli), eqs + jnp.sum(m_eq.astype(jnp.int32))

            lax.fori_loop(0, N // 16, comp_body, (jnp.int32(0), jnp.int32(0)))

            def rank_body(i, _):
                base = (i // 16) * 16
                lane = i - base
                vi = ckey_v[pl.ds(base, 16)]
                ki = jnp.sum(jnp.where(iota == lane, vi, 0))

                def jb(j, acc):
                    kj = ckey_v[pl.ds(j * 16, 16)]
                    lp = iota + j * 16
                    gt = (kj > ki).astype(jnp.int32)
                    eq = ((kj == ki) & (lp < i)).astype(jnp.int32)
                    return acc + gt + eq

                acc = lax.fori_loop(0, k // 16, jb, jnp.zeros((16,), jnp.int32))
                r = jnp.sum(acc)
                pv = cpos_v[pl.ds(base, 16)]
                pos_i = jnp.sum(jnp.where(iota == lane, pv, 0))
                sv = cscr_v[pl.ds(base, 16)]
                scr_i = jnp.sum(jnp.where(iota == lane, sv, jnp.float32(0)))
                mask0 = iota == 0
                ridx = jnp.full((16,), r, jnp.int32)
                plsc.store_scatter(spos_v, [ridx],
                                   jnp.full((16,), pos_i, jnp.int32), mask=mask0)
                plsc.store_scatter(sscr_v, [ridx],
                                   jnp.full((16,), scr_i, jnp.float32), mask=mask0)
                return 0

            lax.fori_loop(0, k, rank_body, 0)
            pltpu.sync_copy(spos_v.at[pl.ds(0, k)], if_out.at[pl.ds(b * k, k)])
            pltpu.sync_copy(sscr_v.at[pl.ds(0, k)], sc_out.at[pl.ds(b * k, k)])

        @pl.when(sid < 4)
        def _():
            unit(K0, sid, sid, sc0_hbm, if0_hbm)

        @pl.when((sid >= 4) & (sid < 8))
        def _():
            unit(K1, sid, sid - 4, sc1_hbm, if1_hbm)

        plsc.subcore_barrier()

        pltpu.sync_copy(if0_hbm.at[pl.ds(wid * r0, r0)], i0_v)
        pltpu.async_copy(xflat_hbm.at[i0_v], rows0_v, sem).wait()
        pltpu.sync_copy(rows0_v, g0_hbm.at[pl.ds(wid * r0, r0)])
        pltpu.sync_copy(if1_hbm.at[pl.ds(wid * r1, r1)], i1_v)
        pltpu.async_copy(xflat_hbm.at[i1_v], rows1_v, sem).wait()
        pltpu.sync_copy(rows1_v, g1_hbm.at[pl.ds(wid * r1, r1)])

    return sck(scores_stk, tkey, tbud, x_flat)


def _mlp_body(g_ref, sc_ref, temb_ref, W1_ref, b1_ref, g1_ref, be1_ref,
              W2_ref, b2_ref, g2_ref, be2_ref, W3_ref, b3_ref, out_ref, *, k):
    gr = g_ref[0]
    sc = sc_ref[0, 0]
    gated = gr * jax.nn.sigmoid(sc)[:, None]
    t = jnp.broadcast_to(temb_ref[0], (k, TASK_DIM))
    hk = jnp.concatenate([gated, t], axis=-1)

    z = hk @ W1_ref[...] + b1_ref[...]
    m = z.mean(-1, keepdims=True)
    v = z.var(-1, keepdims=True)
    z = (z - m) / jnp.sqrt(v + 1e-5) * g1_ref[...] + be1_ref[...]
    z = jax.nn.gelu(z)

    z = z @ W2_ref[...] + b2_ref[...]
    m = z.mean(-1, keepdims=True)
    v = z.var(-1, keepdims=True)
    z = (z - m) / jnp.sqrt(v + 1e-5) * g2_ref[...] + be2_ref[...]
    z = jax.nn.gelu(z)

    phase = jnp.sin((z @ W3_ref[...] + b3_ref[...]) * PHASE_DIV)
    out_ref[0] = jnp.concatenate([gated, phase], axis=-1)


def _mlp(gathered, topk_scores, temb, W1, b1, g1, be1, W2, b2, g2, be2, W3, b3, k):
    ph = W3.shape[-1]
    return pl.pallas_call(
        functools.partial(_mlp_body, k=k),
        grid=(B,),
        in_specs=[
            pl.BlockSpec((1, k, D), lambda b: (b, 0, 0)),
            pl.BlockSpec((1, 1, k), lambda b: (b, 0, 0)),
            pl.BlockSpec((1, 1, TASK_DIM), lambda b: (b, 0, 0)),
            pl.BlockSpec(W1.shape, lambda b: (0, 0)),
            pl.BlockSpec(b1.shape, lambda b: (0,)),
            pl.BlockSpec(g1.shape, lambda b: (0,)),
            pl.BlockSpec(be1.shape, lambda b: (0,)),
            pl.BlockSpec(W2.shape, lambda b: (0, 0)),
            pl.BlockSpec(b2.shape, lambda b: (0,)),
            pl.BlockSpec(g2.shape, lambda b: (0,)),
            pl.BlockSpec(be2.shape, lambda b: (0,)),
            pl.BlockSpec(W3.shape, lambda b: (0, 0)),
            pl.BlockSpec(b3.shape, lambda b: (0,)),
        ],
        out_specs=pl.BlockSpec((1, k, D + ph), lambda b: (b, 0, 0)),
        out_shape=jax.ShapeDtypeStruct((B, k, D + ph), jnp.float32),
    )(gathered, topk_scores.reshape(B, 1, k), temb.reshape(B, 1, TASK_DIM),
      W1, b1, g1, be1, W2, b2, g2, be2, W3, b3)


def kernel(x, task_id, te0, Ws0, bs0, W1_0, b1_0, g1_0, be1_0, W2_0, b2_0,
           g2_0, be2_0, W3_0, b3_0, te1, Ws1, bs1, W1_1, b1_1, g1_1, be1_1,
           W2_1, b2_1, g2_1, be2_1, W3_1, b3_1):
    temb0 = te0[task_id]
    temb1 = te1[task_id]
    s0, s1 = _scores(x, temb0, temb1, Ws0, bs0, Ws1, bs1)
    tkey, tbud = _threshold(s0, s1)

    scores_stk = jnp.concatenate(
        [s0.reshape(B, N), s1.reshape(B, N)], axis=0)
    x_flat = x.reshape(B * N, D)
    sc0, sc1, _if0, _if1, g0, g1 = _sc_select_gather(
        scores_stk, tkey, tbud, x_flat)

    out0 = _mlp(g0.reshape(B, K0, D), sc0.reshape(B, K0), temb0,
                W1_0, b1_0, g1_0, be1_0, W2_0, b2_0, g2_0, be2_0, W3_0, b3_0, K0)
    out1 = _mlp(g1.reshape(B, K1, D), sc1.reshape(B, K1), temb1,
                W1_1, b1_1, g1_1, be1_1, W2_1, b2_1, g2_1, be2_1, W3_1, b3_1, K1)
    return (out0, out1)


# speedup vs baseline: 1.1978x; 1.1978x over previous
import dataclasses
import functools

import jax
import jax.numpy as jnp
from jax import lax
from jax.experimental import pallas as pl
from jax.experimental.pallas import tpu as pltpu
from jax.experimental.pallas import tpu_sc as plsc

B, N, D = 4, 8192, 768
TASK_DIM = 64
PHASE_DIV = 2.0
BN = 2048
K0, K1 = 512, 256
NW = 32


def _score_body(x_ref, temb0_ref, temb1_ref, w0_ref, w1_ref, b0_ref, b1_ref,
                s0_ref, s1_ref):
    xb = x_ref[0]
    t0 = jnp.broadcast_to(temb0_ref[0], (BN, TASK_DIM))
    t1 = jnp.broadcast_to(temb1_ref[0], (BN, TASK_DIM))
    h0 = jnp.concatenate([xb, t0], axis=-1)
    h1 = jnp.concatenate([xb, t1], axis=-1)
    s0 = h0 @ w0_ref[...] + b0_ref[...]
    s1 = h1 @ w1_ref[...] + b1_ref[...]
    s0_ref[0, 0, :] = s0[:, 0]
    s1_ref[0, 0, :] = s1[:, 0]


def _scores(x, temb0, temb1, Ws0, bs0, Ws1, bs1):
    grid = (B, N // BN)
    s0, s1 = pl.pallas_call(
        _score_body,
        grid=grid,
        in_specs=[
            pl.BlockSpec((1, BN, D), lambda b, n: (b, n, 0)),
            pl.BlockSpec((1, 1, TASK_DIM), lambda b, n: (b, 0, 0)),
            pl.BlockSpec((1, 1, TASK_DIM), lambda b, n: (b, 0, 0)),
            pl.BlockSpec((D + TASK_DIM, 1), lambda b, n: (0, 0)),
            pl.BlockSpec((D + TASK_DIM, 1), lambda b, n: (0, 0)),
            pl.BlockSpec((1,), lambda b, n: (0,)),
            pl.BlockSpec((1,), lambda b, n: (0,)),
        ],
        out_specs=[
            pl.BlockSpec((1, 1, BN), lambda b, n: (b, 0, n)),
            pl.BlockSpec((1, 1, BN), lambda b, n: (b, 0, n)),
        ],
        out_shape=[
            jax.ShapeDtypeStruct((B, 1, N), jnp.float32),
            jax.ShapeDtypeStruct((B, 1, N), jnp.float32),
        ],
    )(x, temb0.reshape(B, 1, TASK_DIM), temb1.reshape(B, 1, TASK_DIM),
      Ws0, Ws1, bs0, bs1)
    return s0, s1


def _thresh_body(s0_ref, s1_ref, tkey_ref, tbud_ref):
    rows = []
    for s_ref, k in ((s0_ref, K0), (s1_ref, K1)):
        s = s_ref[:, 0, :]
        bits = lax.bitcast_convert_type(s, jnp.uint32)
        ukey = jnp.where(s >= 0, bits | jnp.uint32(0x80000000), ~bits)
        t = jnp.zeros((B, 1), jnp.uint32)
        for bit in range(31, -1, -1):
            cand = t | jnp.uint32(1 << bit)
            cnt = jnp.sum((ukey >= cand).astype(jnp.int32), axis=1,
                          keepdims=True)
            t = jnp.where(cnt >= k, cand, t)
        cgt = jnp.sum((ukey > t).astype(jnp.int32), axis=1, keepdims=True)
        skey = lax.bitcast_convert_type(t ^ jnp.uint32(0x80000000), jnp.int32)
        rows.append((skey, k - cgt))
    tkey = jnp.concatenate([rows[0][0], rows[1][0]], axis=0)
    tbud = jnp.concatenate([rows[0][1], rows[1][1]], axis=0)
    tkey_ref[...] = jnp.broadcast_to(tkey, (2 * B, 16))
    tbud_ref[...] = jnp.broadcast_to(tbud, (2 * B, 16))


def _threshold(s0, s1):
    return pl.pallas_call(
        _thresh_body,
        out_shape=[
            jax.ShapeDtypeStruct((2 * B, 16), jnp.int32),
            jax.ShapeDtypeStruct((2 * B, 16), jnp.int32),
        ],
    )(s0, s1)


def _skey16(v):
    b = lax.bitcast_convert_type(v, jnp.int32)
    return b ^ ((b >> 31) & jnp.int32(0x7FFFFFFF))


def _sc_select_gather(scores_stk, tkey, tbud, x_flat):
    mesh = plsc.VectorSubcoreMesh(core_axis_name="c", subcore_axis_name="s")
    r0, r1 = K0 * B // NW, K1 * B // NW
    cp = pltpu.CompilerParams()
    if "needs_layout_passes" in pltpu.CompilerParams.__dataclass_fields__:
        cp = dataclasses.replace(cp, needs_layout_passes=False)

    @functools.partial(
        pl.kernel, mesh=mesh, compiler_params=cp,
        out_type=[
            jax.ShapeDtypeStruct((B * K0,), jnp.float32),
            jax.ShapeDtypeStruct((B * K1,), jnp.float32),
            jax.ShapeDtypeStruct((B * K0,), jnp.int32),
            jax.ShapeDtypeStruct((B * K1,), jnp.int32),
            jax.ShapeDtypeStruct((B * K0, D), jnp.float32),
            jax.ShapeDtypeStruct((B * K1, D), jnp.float32),
        ],
        scratch_types=[
            pltpu.VMEM((N,), jnp.float32),
            pltpu.VMEM((16,), jnp.int32),
            pltpu.VMEM((16,), jnp.int32),
            pltpu.VMEM((K0,), jnp.int32),
            pltpu.VMEM((K0,), jnp.int32),
            pltpu.VMEM((K0,), jnp.float32),
            pltpu.VMEM((K0,), jnp.int32),
            pltpu.VMEM((K0,), jnp.float32),
            pltpu.VMEM((r0,), jnp.int32),
            pltpu.VMEM((r1,), jnp.int32),
            pltpu.VMEM((r0, D), jnp.float32),
            pltpu.VMEM((r1, D), jnp.float32),
            pltpu.SemaphoreType.DMA,
        ],
    )
    def sck(scores_hbm, tkey_hbm, tbud_hbm, xflat_hbm,
            sc0_hbm, sc1_hbm, if0_hbm, if1_hbm, g0_hbm, g1_hbm,
            scores_v, tk_v, tb_v, ckey_v, cpos_v, cscr_v, spos_v, sscr_v,
            i0_v, i1_v, rows0_v, rows1_v, sem):
        sid = lax.axis_index("s")
        wid = sid * 2 + lax.axis_index("c")
        iota = lax.iota(jnp.int32, 16)

        def unit(k, row, b, sc_out, if_out):
            pltpu.sync_copy(scores_hbm.at[row], scores_v)
            pltpu.sync_copy(tkey_hbm.at[row], tk_v)
            pltpu.sync_copy(tbud_hbm.at[row], tb_v)
            tkey_vec = tk_v[...]
            bud_vec = tb_v[...]

            def comp_body(n, carry):
                cur, eqs = carry
                v = scores_v[pl.ds(n * 16, 16)]
                key = _skey16(v)
                m_gt = key > tkey_vec
                m_eq = key == tkey_vec
                psum = jnp.cumsum(m_eq.astype(jnp.int32))
                sel = m_gt | (m_eq & ((psum + eqs) <= bud_vec))
                seli = sel.astype(jnp.int32)
                dest = (jnp.cumsum(seli) - seli) + cur
                posv = iota + (n * 16 + b * N)
                plsc.store_scatter(ckey_v, [dest], key, mask=sel)
                plsc.store_scatter(cpos_v, [dest], posv, mask=sel)
                plsc.store_scatter(cscr_v, [dest], v, mask=sel)
                return cur + jnp.sum(seli), eqs + jnp.sum(m_eq.astype(jnp.int32))

            lax.fori_loop(0, N // 16, comp_body, (jnp.int32(0), jnp.int32(0)),
                          unroll=4)

            def rank_body(i, _):
                base = (i // 16) * 16
                lane = i - base
                vi = ckey_v[pl.ds(base, 16)]
                ki = jnp.sum(jnp.where(iota == lane, vi, 0))

                def jb(j, acc):
                    kj = ckey_v[pl.ds(j * 16, 16)]
                    lp = iota + j * 16
                    gt = (kj > ki).astype(jnp.int32)
                    eq = ((kj == ki) & (lp < i)).astype(jnp.int32)
                    return acc + gt + eq

                acc = lax.fori_loop(0, k // 16, jb, jnp.zeros((16,), jnp.int32),
                                    unroll=8)
                r = jnp.sum(acc)
                pv = cpos_v[pl.ds(base, 16)]
                pos_i = jnp.sum(jnp.where(iota == lane, pv, 0))
                sv = cscr_v[pl.ds(base, 16)]
                scr_i = jnp.sum(jnp.where(iota == lane, sv, jnp.float32(0)))
                mask0 = iota == 0
                ridx = jnp.full((16,), r, jnp.int32)
                plsc.store_scatter(spos_v, [ridx],
                                   jnp.full((16,), pos_i, jnp.int32), mask=mask0)
                plsc.store_scatter(sscr_v, [ridx],
                                   jnp.full((16,), scr_i, jnp.float32), mask=mask0)
                return 0

            lax.fori_loop(0, k, rank_body, 0)
            pltpu.sync_copy(spos_v.at[pl.ds(0, k)], if_out.at[pl.ds(b * k, k)])
            pltpu.sync_copy(sscr_v.at[pl.ds(0, k)], sc_out.at[pl.ds(b * k, k)])

        @pl.when(sid < 4)
        def _():
            unit(K0, sid, sid, sc0_hbm, if0_hbm)

        @pl.when((sid >= 4) & (sid < 8))
        def _():
            unit(K1, sid, sid - 4, sc1_hbm, if1_hbm)

        plsc.subcore_barrier()

        pltpu.sync_copy(if0_hbm.at[pl.ds(wid * r0, r0)], i0_v)
        pltpu.sync_copy(if1_hbm.at[pl.ds(wid * r1, r1)], i1_v)
        c0 = pltpu.async_copy(xflat_hbm.at[i0_v], rows0_v, sem)
        c1 = pltpu.async_copy(xflat_hbm.at[i1_v], rows1_v, sem)
        c0.wait()
        pltpu.sync_copy(rows0_v, g0_hbm.at[pl.ds(wid * r0, r0)])
        c1.wait()
        pltpu.sync_copy(rows1_v, g1_hbm.at[pl.ds(wid * r1, r1)])

    return sck(scores_stk, tkey, tbud, x_flat)


def _mlp_body(g_ref, sc_ref, temb_ref, W1_ref, b1_ref, g1_ref, be1_ref,
              W2_ref, b2_ref, g2_ref, be2_ref, W3_ref, b3_ref, out_ref, *, k):
    gr = g_ref[0]
    sc = sc_ref[0, 0]
    gated = gr * jax.nn.sigmoid(sc)[:, None]
    t = jnp.broadcast_to(temb_ref[0], (k, TASK_DIM))
    hk = jnp.concatenate([gated, t], axis=-1)

    z = hk @ W1_ref[...] + b1_ref[...]
    m = z.mean(-1, keepdims=True)
    v = z.var(-1, keepdims=True)
    z = (z - m) / jnp.sqrt(v + 1e-5) * g1_ref[...] + be1_ref[...]
    z = jax.nn.gelu(z)

    z = z @ W2_ref[...] + b2_ref[...]
    m = z.mean(-1, keepdims=True)
    v = z.var(-1, keepdims=True)
    z = (z - m) / jnp.sqrt(v + 1e-5) * g2_ref[...] + be2_ref[...]
    z = jax.nn.gelu(z)

    phase = jnp.sin((z @ W3_ref[...] + b3_ref[...]) * PHASE_DIV)
    out_ref[0] = jnp.concatenate([gated, phase], axis=-1)


def _mlp(gathered, topk_scores, temb, W1, b1, g1, be1, W2, b2, g2, be2, W3, b3, k):
    ph = W3.shape[-1]
    return pl.pallas_call(
        functools.partial(_mlp_body, k=k),
        grid=(B,),
        in_specs=[
            pl.BlockSpec((1, k, D), lambda b: (b, 0, 0)),
            pl.BlockSpec((1, 1, k), lambda b: (b, 0, 0)),
            pl.BlockSpec((1, 1, TASK_DIM), lambda b: (b, 0, 0)),
            pl.BlockSpec(W1.shape, lambda b: (0, 0)),
            pl.BlockSpec(b1.shape, lambda b: (0,)),
            pl.BlockSpec(g1.shape, lambda b: (0,)),
            pl.BlockSpec(be1.shape, lambda b: (0,)),
            pl.BlockSpec(W2.shape, lambda b: (0, 0)),
            pl.BlockSpec(b2.shape, lambda b: (0,)),
            pl.BlockSpec(g2.shape, lambda b: (0,)),
            pl.BlockSpec(be2.shape, lambda b: (0,)),
            pl.BlockSpec(W3.shape, lambda b: (0, 0)),
            pl.BlockSpec(b3.shape, lambda b: (0,)),
        ],
        out_specs=pl.BlockSpec((1, k, D + ph), lambda b: (b, 0, 0)),
        out_shape=jax.ShapeDtypeStruct((B, k, D + ph), jnp.float32),
    )(gathered, topk_scores.reshape(B, 1, k), temb.reshape(B, 1, TASK_DIM),
      W1, b1, g1, be1, W2, b2, g2, be2, W3, b3)


def kernel(x, task_id, te0, Ws0, bs0, W1_0, b1_0, g1_0, be1_0, W2_0, b2_0,
           g2_0, be2_0, W3_0, b3_0, te1, Ws1, bs1, W1_1, b1_1, g1_1, be1_1,
           W2_1, b2_1, g2_1, be2_1, W3_1, b3_1):
    temb0 = te0[task_id]
    temb1 = te1[task_id]
    s0, s1 = _scores(x, temb0, temb1, Ws0, bs0, Ws1, bs1)
    tkey, tbud = _threshold(s0, s1)

    scores_stk = jnp.concatenate(
        [s0.reshape(B, N), s1.reshape(B, N)], axis=0)
    x_flat = x.reshape(B * N, D)
    sc0, sc1, _if0, _if1, g0, g1 = _sc_select_gather(
        scores_stk, tkey, tbud, x_flat)

    out0 = _mlp(g0.reshape(B, K0, D), sc0.reshape(B, K0), temb0,
                W1_0, b1_0, g1_0, be1_0, W2_0, b2_0, g2_0, be2_0, W3_0, b3_0, K0)
    out1 = _mlp(g1.reshape(B, K1, D), sc1.reshape(B, K1), temb1,
                W1_1, b1_1, g1_1, be1_1, W2_1, b2_1, g2_1, be2_1, W3_1, b3_1, K1)
    return (out0, out1)
